# baseline (speedup 1.0000x reference)
import functools

import jax
import jax.numpy as jnp
from jax import lax
from jax.experimental import pallas as pl
from jax.experimental.pallas import tpu as pltpu
from jax.experimental.pallas import tpu_sc as plsc

_N = 4096
_NW = 32
_SLABS_PER_W = _N // _NW
_CHUNK = 4
_NCHUNK = _SLABS_PER_W // _CHUNK

_mesh = plsc.VectorSubcoreMesh(core_axis_name="c", subcore_axis_name="s")


@functools.partial(
    pl.kernel,
    mesh=_mesh,
    out_type=jax.ShapeDtypeStruct((_N, 2, _N), jnp.float32),
    scratch_types=[
        pltpu.VMEM((_CHUNK, 2, _N), jnp.float32),
        pltpu.VMEM((_CHUNK, 2, _N), jnp.float32),
        pltpu.SemaphoreType.DMA,
        pltpu.SemaphoreType.DMA,
        pltpu.SemaphoreType.DMA,
        pltpu.SemaphoreType.DMA,
    ],
)
def _sc_copy(in_hbm, out_hbm, buf0, buf1, si0, si1, so0, so1):
    wid = lax.axis_index("s") * 2 + lax.axis_index("c")
    base = wid * _SLABS_PER_W
    bufs = (buf0, buf1)
    sins = (si0, si1)
    souts = (so0, so1)

    def in_copy(c):
        sl = pl.ds(base + c * _CHUNK, _CHUNK)
        return pltpu.make_async_copy(in_hbm.at[sl], bufs[c % 2], sins[c % 2])

    def out_copy(c):
        sl = pl.ds(base + c * _CHUNK, _CHUNK)
        return pltpu.make_async_copy(bufs[c % 2], out_hbm.at[sl], souts[c % 2])

    in_copy(0).start()
    for c in range(_NCHUNK):
        in_copy(c).wait()
        out_copy(c).start()
        if c + 1 < _NCHUNK:
            if c >= 1:
                out_copy(c - 1).wait()
            in_copy(c + 1).start()
    out_copy(_NCHUNK - 1).wait()


def kernel(inputs, w):
    del inputs
    x = jnp.transpose(w, (0, 2, 1))
    y = _sc_copy(x)
    return jnp.transpose(y, (0, 2, 1))


# speedup vs baseline: 1.0136x; 1.0136x over previous
import functools

import jax
import jax.numpy as jnp
from jax import lax
from jax.experimental import pallas as pl
from jax.experimental.pallas import tpu as pltpu
from jax.experimental.pallas import tpu_sc as plsc

_N = 4096
_NW = 32
_SLABS_PER_W = _N // _NW
_CHUNK = 4
_NBUF = 3
_NCHUNK = _SLABS_PER_W // _CHUNK

_mesh = plsc.VectorSubcoreMesh(core_axis_name="c", subcore_axis_name="s")


@functools.partial(
    pl.kernel,
    mesh=_mesh,
    out_type=jax.ShapeDtypeStruct((_N, 2, _N), jnp.float32),
    scratch_types=(
        [pltpu.VMEM((_CHUNK, 2, _N), jnp.float32)] * _NBUF
        + [pltpu.SemaphoreType.DMA] * (2 * _NBUF)
    ),
)
def _sc_copy(in_hbm, out_hbm, *scratch):
    wid = lax.axis_index("s") * 2 + lax.axis_index("c")
    base = wid * _SLABS_PER_W
    bufs = scratch[:_NBUF]
    sins = scratch[_NBUF:2 * _NBUF]
    souts = scratch[2 * _NBUF:]

    def in_copy(c):
        sl = pl.ds(base + c * _CHUNK, _CHUNK)
        b = c % _NBUF
        return pltpu.make_async_copy(in_hbm.at[sl], bufs[b], sins[b])

    def out_copy(c):
        sl = pl.ds(base + c * _CHUNK, _CHUNK)
        b = c % _NBUF
        return pltpu.make_async_copy(bufs[b], out_hbm.at[sl], souts[b])

    for c in range(_NBUF - 1):
        in_copy(c).start()
    for c in range(_NCHUNK):
        in_copy(c).wait()
        out_copy(c).start()
        nxt = c + _NBUF - 1
        if nxt < _NCHUNK:
            if nxt >= _NBUF:
                out_copy(nxt - _NBUF).wait()
            in_copy(nxt).start()
    for c in range(_NCHUNK - _NBUF, _NCHUNK):
        if c >= 0:
            out_copy(c).wait()


def kernel(inputs, w):
    del inputs
    x = jnp.transpose(w, (0, 2, 1))
    y = _sc_copy(x)
    return jnp.transpose(y, (0, 2, 1))


# speedup vs baseline: 1.3928x; 1.3741x over previous
import jax
import jax.numpy as jnp
from jax.experimental import pallas as pl
from jax.experimental.pallas import tpu as pltpu

_N = 4096
_CHUNK = 256
_NBUF = 4
_NCHUNK = _N // _CHUNK


def _dma_body(in_hbm, out_hbm, *scratch):
    bufs = scratch[:_NBUF]
    sins = scratch[_NBUF:2 * _NBUF]
    souts = scratch[2 * _NBUF:]

    def in_copy(c):
        sl = pl.ds(c * _CHUNK, _CHUNK)
        b = c % _NBUF
        return pltpu.make_async_copy(in_hbm.at[sl], bufs[b], sins[b])

    def out_copy(c):
        sl = pl.ds(c * _CHUNK, _CHUNK)
        b = c % _NBUF
        return pltpu.make_async_copy(bufs[b], out_hbm.at[sl], souts[b])

    for c in range(_NBUF - 1):
        in_copy(c).start()
    for c in range(_NCHUNK):
        in_copy(c).wait()
        out_copy(c).start()
        nxt = c + _NBUF - 1
        if nxt < _NCHUNK:
            if nxt >= _NBUF:
                out_copy(nxt - _NBUF).wait()
            in_copy(nxt).start()
    for c in range(max(0, _NCHUNK - _NBUF), _NCHUNK):
        out_copy(c).wait()


def kernel(inputs, w):
    del inputs
    x = jnp.transpose(w, (0, 2, 1))
    y = pl.pallas_call(
        _dma_body,
        in_specs=[pl.BlockSpec(memory_space=pl.ANY)],
        out_specs=pl.BlockSpec(memory_space=pl.ANY),
        out_shape=jax.ShapeDtypeStruct((_N, 2, _N), jnp.float32),
        scratch_shapes=(
            [pltpu.VMEM((_CHUNK, 2, _N), jnp.float32)] * _NBUF
            + [pltpu.SemaphoreType.DMA] * (2 * _NBUF)
        ),
    )(x)
    return jnp.transpose(y, (0, 2, 1))
